# baseline (speedup 1.0000x reference)
import functools

import jax
import jax.numpy as jnp
from jax import lax
from jax.experimental import pallas as pl
from jax.experimental.pallas import tpu as pltpu
from jax.experimental.pallas import tpu_sc as plsc

N0 = 10000
NP = 10240
D = 128
K = 8
BQ = 256
NSUB = NP // 128
TL = 4

SC_NC = 2
SC_NS = 16
SC_NW = SC_NC * SC_NS
GATHER_CHUNK = 512


def _knn_proj_body(x_ref, c4_ref, c3t_ref, w1a_ref, w1b_ref, w1c_ref, b1_ref,
                   idx_ref, qv_ref, kv_ref):
    c4 = c4_ref[...]
    c3t = c3t_ref[...]
    lane8 = lax.broadcasted_iota(jnp.int32, (BQ, 8), 1)
    c3q = jnp.where(lane8 < 3, c4, 0.0)
    sq_q = jnp.sum(c3q * c3q, axis=1, keepdims=True)
    sq_k = jnp.sum(c3t * c3t, axis=0, keepdims=True)
    col1 = lax.broadcasted_iota(jnp.int32, (1, NP), 1)
    sq_k = jnp.where(col1 < N0, sq_k, 1e30)
    qaug = jnp.where(lane8 < 3, -2.0 * c3q,
                     jnp.where(lane8 == 3, sq_q, 0.0))
    r8 = lax.broadcasted_iota(jnp.int32, (8, NP), 0)
    kaug = jnp.where(r8 < 3, c3t, jnp.where(r8 == 3, 1.0, 0.0))
    d2 = (jnp.dot(qaug, kaug, preferred_element_type=jnp.float32)
          + sq_k)

    lane = lax.broadcasted_iota(jnp.int32, (BQ, 128), 1)
    mk = [jnp.full((BQ, 128), jnp.inf, jnp.float32) for _ in range(TL)]
    for s in range(NSUB):
        tv = jnp.maximum(d2[:, s * 128:(s + 1) * 128], 1e-30)
        t = lax.bitcast_convert_type(
            (lax.bitcast_convert_type(tv, jnp.int32)
             & jnp.int32(~0x7F)) | jnp.int32(s), jnp.float32)
        for i in range(TL):
            mk[i], t = jnp.minimum(t, mk[i]), jnp.maximum(t, mk[i])
    ck = jnp.concatenate(mk, axis=1)
    cc = jnp.concatenate(
        [(lax.bitcast_convert_type(k, jnp.int32) & jnp.int32(0x7F)) * 128
         + lane for k in mk], axis=1)
    idx_cols = []
    for _ in range(K):
        m = jnp.min(ck, axis=1, keepdims=True)
        sel = jnp.min(jnp.where(ck == m, cc, NP), axis=1, keepdims=True)
        idx_cols.append(sel)
        ck = jnp.where((ck == m) & (cc == sel), jnp.inf, ck)
    idx_ref[...] = jnp.concatenate(idx_cols, axis=1)

    x = x_ref[...]
    c_w1c = jnp.dot(c4, w1c_ref[...], preferred_element_type=jnp.float32)
    qv_ref[...] = (jnp.dot(x, w1a_ref[...], preferred_element_type=jnp.float32)
                   - c_w1c + b1_ref[...])
    kv_ref[...] = (jnp.dot(x, w1b_ref[...], preferred_element_type=jnp.float32)
                   + c_w1c)


def _knn_proj(xp, c4p, c3t, w1a, w1b, w1c, b1):
    return pl.pallas_call(
        _knn_proj_body,
        grid=(NP // BQ,),
        in_specs=[
            pl.BlockSpec((BQ, D), lambda i: (i, 0)),
            pl.BlockSpec((BQ, 8), lambda i: (i, 0)),
            pl.BlockSpec((8, NP), lambda i: (0, 0)),
            pl.BlockSpec((D, D), lambda i: (0, 0)),
            pl.BlockSpec((D, D), lambda i: (0, 0)),
            pl.BlockSpec((8, D), lambda i: (0, 0)),
            pl.BlockSpec((1, D), lambda i: (0, 0)),
        ],
        out_specs=[
            pl.BlockSpec((BQ, K), lambda i: (i, 0)),
            pl.BlockSpec((BQ, D), lambda i: (i, 0)),
            pl.BlockSpec((BQ, D), lambda i: (i, 0)),
        ],
        out_shape=[
            jax.ShapeDtypeStruct((NP, K), jnp.int32),
            jax.ShapeDtypeStruct((NP, D), jnp.float32),
            jax.ShapeDtypeStruct((NP, D), jnp.float32),
        ],
        compiler_params=pltpu.CompilerParams(
            dimension_semantics=("parallel",)),
    )(xp, c4p, c3t, w1a, w1b, w1c, b1)


def _sc_gather(table, idx_flat):
    b_total = idx_flat.shape[0]
    b_per_w = b_total // SC_NW
    mesh = plsc.VectorSubcoreMesh(core_axis_name="c", subcore_axis_name="s")

    @functools.partial(
        pl.kernel, mesh=mesh,
        out_type=jax.ShapeDtypeStruct((b_total, D), jnp.float32),
        scratch_types=[
            pltpu.VMEM((GATHER_CHUNK,), jnp.int32),
            pltpu.VMEM((GATHER_CHUNK, D), jnp.float32),
            pltpu.SemaphoreType.DMA,
        ],
    )
    def k(table_hbm, idx_hbm, out_hbm, idx_v, rows_v, sem):
        wid = lax.axis_index("s") * SC_NC + lax.axis_index("c")
        base = wid * b_per_w

        @pl.loop(0, b_per_w, step=GATHER_CHUNK)
        def _(off):
            pltpu.sync_copy(idx_hbm.at[pl.ds(base + off, GATHER_CHUNK)], idx_v)
            pltpu.async_copy(table_hbm.at[idx_v], rows_v, sem).wait()
            pltpu.sync_copy(rows_v, out_hbm.at[pl.ds(base + off, GATHER_CHUNK)])

    return k(table, idx_flat)


def _msg_update_body(qv_ref, g_ref, x_ref, wm2_ref, bm2_ref, wu1a_ref,
                     wu1b_ref, bu1_ref, wu2_ref, bu2_ref, o_ref):
    q3 = qv_ref[...].reshape(BQ, 1, D)
    g3 = g_ref[...].reshape(BQ, K, D)
    h = jnp.maximum(q3 + g3, 0.0).reshape(BQ * K, D)
    msg = (jnp.dot(h, wm2_ref[...], preferred_element_type=jnp.float32)
           + bm2_ref[...])
    aggr = jnp.max(msg.reshape(BQ, K, D), axis=1)
    t = jnp.maximum(
        jnp.dot(x_ref[...], wu1a_ref[...], preferred_element_type=jnp.float32)
        + jnp.dot(aggr, wu1b_ref[...], preferred_element_type=jnp.float32)
        + bu1_ref[...], 0.0)
    o_ref[...] = (jnp.dot(t, wu2_ref[...], preferred_element_type=jnp.float32)
                  + bu2_ref[...])


def _msg_update(qv, g, xp, wm2, bm2, wu1a, wu1b, bu1, wu2, bu2):
    return pl.pallas_call(
        _msg_update_body,
        grid=(NP // BQ,),
        in_specs=[
            pl.BlockSpec((BQ, D), lambda i: (i, 0)),
            pl.BlockSpec((BQ * K, D), lambda i: (i, 0)),
            pl.BlockSpec((BQ, D), lambda i: (i, 0)),
            pl.BlockSpec((D, D), lambda i: (0, 0)),
            pl.BlockSpec((1, D), lambda i: (0, 0)),
            pl.BlockSpec((D, D), lambda i: (0, 0)),
            pl.BlockSpec((D, D), lambda i: (0, 0)),
            pl.BlockSpec((1, D), lambda i: (0, 0)),
            pl.BlockSpec((D, D), lambda i: (0, 0)),
            pl.BlockSpec((1, D), lambda i: (0, 0)),
        ],
        out_specs=pl.BlockSpec((BQ, D), lambda i: (i, 0)),
        out_shape=jax.ShapeDtypeStruct((NP, D), jnp.float32),
        compiler_params=pltpu.CompilerParams(
            dimension_semantics=("parallel",)),
    )(qv, g, xp, wm2, bm2, wu1a, wu1b, bu1, wu2, bu2)


def kernel(x, coords, W_m1, b_m1, W_m2, b_m2, W_u1, b_u1, W_u2, b_u2):
    xp = jnp.pad(x, ((0, NP - N0), (0, 0)))
    c4p = jnp.pad(coords, ((0, NP - N0), (0, 8 - coords.shape[1])))
    c3t = jnp.pad(coords[:, :3].T, ((0, 5), (0, NP - N0)))
    w1a = W_m1[:D]
    w1b = W_m1[D:2 * D]
    w1c = jnp.pad(W_m1[2 * D:], ((0, 4), (0, 0)))
    b1 = b_m1.reshape(1, D)

    idx, qv, kv = _knn_proj(xp, c4p, c3t, w1a, w1b, w1c, b1)
    g = _sc_gather(kv, idx.reshape(NP * K))
    out = _msg_update(qv, g, xp, W_m2, b_m2.reshape(1, D),
                      W_u1[:D], W_u1[D:], b_u1.reshape(1, D),
                      W_u2, b_u2.reshape(1, D))
    return out[:N0]


# speedup vs baseline: 1.0849x; 1.0849x over previous
import functools

import jax
import jax.numpy as jnp
from jax import lax
from jax.experimental import pallas as pl
from jax.experimental.pallas import tpu as pltpu
from jax.experimental.pallas import tpu_sc as plsc

N0 = 10000
NP = 10240
D = 128
K = 8
BQ = 256
NSUB = NP // 128
TL = 3

SC_NC = 2
SC_NS = 16
SC_NW = SC_NC * SC_NS
GATHER_CHUNK = 512


def _knn_proj_body(x_ref, c4_ref, c3t_ref, w1a_ref, w1b_ref, w1c_ref, b1_ref,
                   idx_ref, qv_ref, kv_ref):
    c4 = c4_ref[...]
    c3t = c3t_ref[...]
    lane8 = lax.broadcasted_iota(jnp.int32, (BQ, 8), 1)
    c3q = jnp.where(lane8 < 3, c4, 0.0)
    sq_q = jnp.sum(c3q * c3q, axis=1, keepdims=True)
    sq_k = jnp.sum(c3t * c3t, axis=0, keepdims=True)
    col1 = lax.broadcasted_iota(jnp.int32, (1, NP), 1)
    sq_k = jnp.where(col1 < N0, sq_k, 1e30)
    qaug = jnp.where(lane8 < 3, -2.0 * c3q,
                     jnp.where(lane8 == 3, sq_q, 0.0))
    r8 = lax.broadcasted_iota(jnp.int32, (8, NP), 0)
    kaug = jnp.where(r8 < 3, c3t, jnp.where(r8 == 3, 1.0, 0.0))
    d2 = (jnp.dot(qaug, kaug, preferred_element_type=jnp.float32)
          + sq_k)

    lane = lax.broadcasted_iota(jnp.int32, (BQ, 128), 1)
    mk = [jnp.full((BQ, 128), jnp.inf, jnp.float32) for _ in range(TL)]
    for s in range(NSUB):
        tv = jnp.maximum(d2[:, s * 128:(s + 1) * 128], 1e-30)
        t = lax.bitcast_convert_type(
            (lax.bitcast_convert_type(tv, jnp.int32)
             & jnp.int32(~0x7F)) | jnp.int32(s), jnp.float32)
        for i in range(TL):
            mk[i], t = jnp.minimum(t, mk[i]), jnp.maximum(t, mk[i])
    ck = jnp.concatenate(mk, axis=1)
    cc = jnp.concatenate(
        [(lax.bitcast_convert_type(k, jnp.int32) & jnp.int32(0x7F)) * 128
         + lane for k in mk], axis=1)
    idx_cols = []
    for _ in range(K):
        m = jnp.min(ck, axis=1, keepdims=True)
        sel = jnp.min(jnp.where(ck == m, cc, NP), axis=1, keepdims=True)
        idx_cols.append(sel)
        ck = jnp.where((ck == m) & (cc == sel), jnp.inf, ck)
    idx_ref[...] = jnp.concatenate(idx_cols, axis=1)

    x = x_ref[...]
    c_w1c = jnp.dot(c4, w1c_ref[...], preferred_element_type=jnp.float32)
    qv_ref[...] = (jnp.dot(x, w1a_ref[...], preferred_element_type=jnp.float32)
                   - c_w1c + b1_ref[...])
    kv_ref[...] = (jnp.dot(x, w1b_ref[...], preferred_element_type=jnp.float32)
                   + c_w1c)


def _knn_proj(xp, c4p, c3t, w1a, w1b, w1c, b1):
    return pl.pallas_call(
        _knn_proj_body,
        grid=(NP // BQ,),
        in_specs=[
            pl.BlockSpec((BQ, D), lambda i: (i, 0)),
            pl.BlockSpec((BQ, 8), lambda i: (i, 0)),
            pl.BlockSpec((8, NP), lambda i: (0, 0)),
            pl.BlockSpec((D, D), lambda i: (0, 0)),
            pl.BlockSpec((D, D), lambda i: (0, 0)),
            pl.BlockSpec((8, D), lambda i: (0, 0)),
            pl.BlockSpec((1, D), lambda i: (0, 0)),
        ],
        out_specs=[
            pl.BlockSpec((BQ, K), lambda i: (i, 0)),
            pl.BlockSpec((BQ, D), lambda i: (i, 0)),
            pl.BlockSpec((BQ, D), lambda i: (i, 0)),
        ],
        out_shape=[
            jax.ShapeDtypeStruct((NP, K), jnp.int32),
            jax.ShapeDtypeStruct((NP, D), jnp.float32),
            jax.ShapeDtypeStruct((NP, D), jnp.float32),
        ],
        compiler_params=pltpu.CompilerParams(
            dimension_semantics=("parallel",)),
    )(xp, c4p, c3t, w1a, w1b, w1c, b1)


def _sc_gather(table, idx_flat):
    b_total = idx_flat.shape[0]
    b_per_w = b_total // SC_NW
    mesh = plsc.VectorSubcoreMesh(core_axis_name="c", subcore_axis_name="s")

    @functools.partial(
        pl.kernel, mesh=mesh,
        out_type=jax.ShapeDtypeStruct((b_total, D), jnp.float32),
        scratch_types=[
            pltpu.VMEM((GATHER_CHUNK,), jnp.int32),
            pltpu.VMEM((GATHER_CHUNK, D), jnp.float32),
            pltpu.SemaphoreType.DMA,
        ],
    )
    def k(table_hbm, idx_hbm, out_hbm, idx_v, rows_v, sem):
        wid = lax.axis_index("s") * SC_NC + lax.axis_index("c")
        base = wid * b_per_w

        @pl.loop(0, b_per_w, step=GATHER_CHUNK)
        def _(off):
            pltpu.sync_copy(idx_hbm.at[pl.ds(base + off, GATHER_CHUNK)], idx_v)
            pltpu.async_copy(table_hbm.at[idx_v], rows_v, sem).wait()
            pltpu.sync_copy(rows_v, out_hbm.at[pl.ds(base + off, GATHER_CHUNK)])

    return k(table, idx_flat)


def _msg_update_body(qv_ref, g_ref, x_ref, wm2_ref, bm2_ref, wu1a_ref,
                     wu1b_ref, bu1_ref, wu2_ref, bu2_ref, o_ref):
    q3 = qv_ref[...].reshape(BQ, 1, D)
    g3 = g_ref[...].reshape(BQ, K, D)
    h = jnp.maximum(q3 + g3, 0.0).reshape(BQ * K, D)
    msg = (jnp.dot(h, wm2_ref[...], preferred_element_type=jnp.float32)
           + bm2_ref[...])
    aggr = jnp.max(msg.reshape(BQ, K, D), axis=1)
    t = jnp.maximum(
        jnp.dot(x_ref[...], wu1a_ref[...], preferred_element_type=jnp.float32)
        + jnp.dot(aggr, wu1b_ref[...], preferred_element_type=jnp.float32)
        + bu1_ref[...], 0.0)
    o_ref[...] = (jnp.dot(t, wu2_ref[...], preferred_element_type=jnp.float32)
                  + bu2_ref[...])


def _msg_update(qv, g, xp, wm2, bm2, wu1a, wu1b, bu1, wu2, bu2):
    return pl.pallas_call(
        _msg_update_body,
        grid=(NP // BQ,),
        in_specs=[
            pl.BlockSpec((BQ, D), lambda i: (i, 0)),
            pl.BlockSpec((BQ * K, D), lambda i: (i, 0)),
            pl.BlockSpec((BQ, D), lambda i: (i, 0)),
            pl.BlockSpec((D, D), lambda i: (0, 0)),
            pl.BlockSpec((1, D), lambda i: (0, 0)),
            pl.BlockSpec((D, D), lambda i: (0, 0)),
            pl.BlockSpec((D, D), lambda i: (0, 0)),
            pl.BlockSpec((1, D), lambda i: (0, 0)),
            pl.BlockSpec((D, D), lambda i: (0, 0)),
            pl.BlockSpec((1, D), lambda i: (0, 0)),
        ],
        out_specs=pl.BlockSpec((BQ, D), lambda i: (i, 0)),
        out_shape=jax.ShapeDtypeStruct((NP, D), jnp.float32),
        compiler_params=pltpu.CompilerParams(
            dimension_semantics=("parallel",)),
    )(qv, g, xp, wm2, bm2, wu1a, wu1b, bu1, wu2, bu2)


def kernel(x, coords, W_m1, b_m1, W_m2, b_m2, W_u1, b_u1, W_u2, b_u2):
    xp = jnp.pad(x, ((0, NP - N0), (0, 0)))
    c4p = jnp.pad(coords, ((0, NP - N0), (0, 8 - coords.shape[1])))
    c3t = jnp.pad(coords[:, :3].T, ((0, 5), (0, NP - N0)))
    w1a = W_m1[:D]
    w1b = W_m1[D:2 * D]
    w1c = jnp.pad(W_m1[2 * D:], ((0, 4), (0, 0)))
    b1 = b_m1.reshape(1, D)

    idx, qv, kv = _knn_proj(xp, c4p, c3t, w1a, w1b, w1c, b1)
    g = _sc_gather(kv, idx.reshape(NP * K))
    out = _msg_update(qv, g, xp, W_m2, b_m2.reshape(1, D),
                      W_u1[:D], W_u1[D:], b_u1.reshape(1, D),
                      W_u2, b_u2.reshape(1, D))
    return out[:N0]
